# baseline (speedup 1.0000x reference)
import functools

import jax
import jax.numpy as jnp
from jax import lax
from jax.experimental import pallas as pl
from jax.experimental.pallas import tpu as pltpu
from jax.experimental.pallas import tpu_sc as plsc

D_MODEL = 64
SCALE = 8.0
C = 128
P = 129


def kernel(x, table):
    NB, NT = x.shape
    V = table.shape[0]

    info = plsc.get_sparse_core_info()
    NC, NS = info.num_cores, info.num_subcores
    NW = NC * NS
    assert NB == C * NW, (NB, C, NW)
    assert NT % 2 == 0

    xT = jnp.transpose(x).astype(jnp.int32)
    table2 = jnp.reshape(table, (V // 2, 2 * D_MODEL))

    mesh = plsc.VectorSubcoreMesh(core_axis_name="c", subcore_axis_name="s")

    @functools.partial(
        pl.kernel,
        mesh=mesh,
        out_type=jax.ShapeDtypeStruct((NT, D_MODEL, NB), jnp.float32),
        compiler_params=pltpu.CompilerParams(needs_layout_passes=False),
        scratch_types=[
            pltpu.VMEM((NT, C), jnp.int32),
            pltpu.VMEM((2, C), jnp.int32),
            pltpu.VMEM((2, C, 2 * D_MODEL), jnp.float32),
            pltpu.VMEM((2, D_MODEL, P), jnp.float32),
            pltpu.SemaphoreType.DMA,
            pltpu.SemaphoreType.DMA,
            pltpu.SemaphoreType.DMA,
            pltpu.SemaphoreType.DMA,
        ],
    )
    def emb(x_hbm, table_hbm, out_hbm, idx_all, kbuf, pairs, outb,
            gsem0, gsem1, osem0, osem1):
        wid = lax.axis_index("c") * NS + lax.axis_index("s")
        b0 = wid * C

        pltpu.sync_copy(x_hbm.at[:, pl.ds(b0, C)], idx_all)

        gsems = (gsem0, gsem1)
        osems = (osem0, osem1)

        def prep(n, b):
            for s in range(C // 16):
                sl = pl.ds(s * 16, 16)
                kbuf[b, sl] = lax.shift_right_logical(idx_all[n, sl], 1)

        def gather_start(b):
            pltpu.make_async_copy(
                table_hbm.at[kbuf.at[b]], pairs.at[b], gsems[b]
            ).start()

        def gather_wait(b):
            pltpu.make_async_copy(
                table_hbm.at[kbuf.at[b]], pairs.at[b], gsems[b]
            ).wait()

        def store_start(n, b):
            pltpu.make_async_copy(
                outb.at[b, :, pl.ds(0, C)],
                out_hbm.at[n, :, pl.ds(b0, C)], osems[b]
            ).start()

        def store_wait(n, b):
            pltpu.make_async_copy(
                outb.at[b, :, pl.ds(0, C)],
                out_hbm.at[n, :, pl.ds(b0, C)], osems[b]
            ).wait()

        prep(0, 0)
        gather_start(0)

        lanes = lax.iota(jnp.int32, 16)

        def outer(i, _):
            n0 = i * 2
            for b in range(2):
                n = n0 + b
                nxt = n + 1

                @pl.when(nxt < NT)
                def _():
                    prep(nxt, 1 - b)
                    gather_start(1 - b)

                gather_wait(b)

                @pl.when(n >= 2)
                def _():
                    store_wait(n - 2, b)

                bvec = jnp.broadcast_to(b, (16,))
                nvec = jnp.broadcast_to(n, (16,))

                @plsc.parallel_loop(0, C, unroll=8)
                def _(r):
                    rvec = jnp.broadcast_to(r, (16,))
                    iv = plsc.load_gather(idx_all, [nvec, rvec])
                    colb = lax.bitwise_and(iv, 1) * D_MODEL + lanes
                    for m in range(D_MODEL // 16):
                        vals = plsc.load_gather(
                            pairs, [bvec, rvec, colb + m * 16]
                        ) * SCALE
                        plsc.store_scatter(
                            outb, [bvec, m * 16 + lanes, rvec], vals
                        )

                store_start(n, b)
            return 0

        lax.fori_loop(0, NT // 2, outer, 0)
        store_wait(NT - 2, 0)
        store_wait(NT - 1, 1)

    outT = emb(xT, table2)
    return jnp.transpose(outT, (2, 0, 1))


# speedup vs baseline: 1.0120x; 1.0120x over previous
import functools

import jax
import jax.numpy as jnp
from jax import lax
from jax.experimental import pallas as pl
from jax.experimental.pallas import tpu as pltpu
from jax.experimental.pallas import tpu_sc as plsc

D_MODEL = 64
SCALE = 8.0
C = 128
P = 129


def kernel(x, table):
    NB, NT = x.shape
    V = table.shape[0]

    info = plsc.get_sparse_core_info()
    NC, NS = info.num_cores, info.num_subcores
    NW = NC * NS
    assert NB == C * NW, (NB, C, NW)
    assert NT % 2 == 0

    xT = jnp.transpose(x).astype(jnp.int32)
    table2 = jnp.reshape(table, (V // 2, 2 * D_MODEL))

    mesh = plsc.VectorSubcoreMesh(core_axis_name="c", subcore_axis_name="s")

    @functools.partial(
        pl.kernel,
        mesh=mesh,
        out_type=jax.ShapeDtypeStruct((NT, D_MODEL, NB), jnp.float32),
        compiler_params=pltpu.CompilerParams(needs_layout_passes=False),
        scratch_types=[
            pltpu.VMEM((NT, C), jnp.int32),
            pltpu.VMEM((C,), jnp.int32),
            pltpu.VMEM((C,), jnp.int32),
            pltpu.VMEM((C, 2 * D_MODEL), jnp.float32),
            pltpu.VMEM((C, 2 * D_MODEL), jnp.float32),
            pltpu.VMEM((D_MODEL, P), jnp.float32),
            pltpu.VMEM((D_MODEL, P), jnp.float32),
            pltpu.SemaphoreType.DMA,
            pltpu.SemaphoreType.DMA,
            pltpu.SemaphoreType.DMA,
            pltpu.SemaphoreType.DMA,
        ],
    )
    def emb(x_hbm, table_hbm, out_hbm, idx_all, kbuf0, kbuf1, pairs0, pairs1,
            outb0, outb1, gsem0, gsem1, osem0, osem1):
        kbufs = (kbuf0, kbuf1)
        pairsb = (pairs0, pairs1)
        outbs = (outb0, outb1)
        wid = lax.axis_index("c") * NS + lax.axis_index("s")
        b0 = wid * C

        pltpu.sync_copy(x_hbm.at[:, pl.ds(b0, C)], idx_all)

        gsems = (gsem0, gsem1)
        osems = (osem0, osem1)

        def prep(n, b):
            for s in range(C // 16):
                sl = pl.ds(s * 16, 16)
                kbufs[b][sl] = lax.shift_right_logical(idx_all[n, sl], 1)

        def gather_start(b):
            pltpu.make_async_copy(
                table_hbm.at[kbufs[b]], pairsb[b], gsems[b]
            ).start()

        def gather_wait(b):
            pltpu.make_async_copy(
                table_hbm.at[kbufs[b]], pairsb[b], gsems[b]
            ).wait()

        def store_start(n, b):
            pltpu.make_async_copy(
                outbs[b].at[:, pl.ds(0, C)],
                out_hbm.at[n, :, pl.ds(b0, C)], osems[b]
            ).start()

        def store_wait(n, b):
            pltpu.make_async_copy(
                outbs[b].at[:, pl.ds(0, C)],
                out_hbm.at[n, :, pl.ds(b0, C)], osems[b]
            ).wait()

        prep(0, 0)
        gather_start(0)

        lanes = lax.iota(jnp.int32, 16)

        def outer(i, _):
            n0 = i * 2
            for b in range(2):
                n = n0 + b
                nxt = n + 1

                @pl.when(nxt < NT)
                def _():
                    prep(nxt, 1 - b)
                    gather_start(1 - b)

                gather_wait(b)

                @pl.when(n >= 2)
                def _():
                    store_wait(n - 2, b)

                nvec = jnp.broadcast_to(n, (16,))
                pairs_b = pairsb[b]
                outb_b = outbs[b]

                @plsc.parallel_loop(0, C, unroll=4)
                def _(r):
                    rvec = jnp.broadcast_to(r, (16,))
                    iv = plsc.load_gather(idx_all, [nvec, rvec])
                    colb = lax.bitwise_and(iv, 1) * D_MODEL + lanes
                    for m in range(D_MODEL // 16):
                        vals = plsc.load_gather(
                            pairs_b, [rvec, colb + m * 16]
                        ) * SCALE
                        plsc.store_scatter(
                            outb_b, [m * 16 + lanes, rvec], vals
                        )

                store_start(n, b)
            return 0

        lax.fori_loop(0, NT // 2, outer, 0)
        store_wait(NT - 2, 0)
        store_wait(NT - 1, 1)

    outT = emb(xT, table2)
    return jnp.transpose(outT, (2, 0, 1))
